# Initial kernel scaffold; baseline (speedup 1.0000x reference)
import jax
import jax.numpy as jnp
from jax.experimental import pallas as pl


def kernel(x, edge_index, W_hp, b_hp, W_lp, b_lp, W_i, b_i, wh, bh, wl, bl, wi, bi):
    raise NotImplementedError("write your pallas kernel here")


# speedup vs baseline: 15.8187x; 15.8187x over previous
import functools

import jax
import jax.numpy as jnp
from jax import lax
from jax.experimental import pallas as pl
from jax.experimental.pallas import tpu as pltpu
from jax.experimental.pallas import tpu_sc as plsc

N = 10000
D = 256
NC, NS, L = 2, 16, 16
NW = NC * NS
NPAD = 10240
ZROW = N
B = 128
F = 144
ROWS_PER_TILE = NPAD // NS


def _deg_body(rowp_hbm, colp_hbm, out_hbm, rv, cv, dloc):
    c = lax.axis_index("c")
    s = lax.axis_index("s")
    wid = s * NC + c
    ed = rv.shape[0]
    pltpu.sync_copy(rowp_hbm.at[wid], rv)
    pltpu.sync_copy(colp_hbm.at[wid], cv)

    zeros = jnp.zeros((L,), jnp.float32)

    def zb(i, carry):
        dloc[pl.ds(i * L, L)] = zeros
        return carry

    lax.fori_loop(0, NPAD // L, zb, 0)

    ones = jnp.ones((L,), jnp.float32)

    def body(i, carry):
        r = rv[pl.ds(i * L, L)]
        cc = cv[pl.ds(i * L, L)]
        plsc.addupdate_scatter(dloc, [cc], ones, mask=r != ZROW)
        return carry

    lax.fori_loop(0, ed // L, body, 0)

    pltpu.sync_copy(dloc, out_hbm.at[wid])


def _make_deg_kernel(ed):
    return pl.kernel(
        _deg_body,
        out_type=jax.ShapeDtypeStruct((NW, NPAD), jnp.float32),
        mesh=plsc.VectorSubcoreMesh(core_axis_name="c", subcore_axis_name="s"),
        compiler_params=pltpu.CompilerParams(needs_layout_passes=False, use_tc_tiling_on_sc=False),
        scratch_types=[
            pltpu.VMEM((ed,), jnp.int32),
            pltpu.VMEM((ed,), jnp.int32),
            pltpu.VMEM((NPAD,), jnp.float32),
        ],
    )


def _prop_body(rowp_hbm, colp_hbm, zlo_hbm, zhi_hbm, outlo_hbm, outhi_hbm,
               rv, cv, buf, acc):
    c = lax.axis_index("c")
    s = lax.axis_index("s")
    ch = rv.shape[0]

    def run(z_ref, out_ref):
        pltpu.sync_copy(rowp_hbm.at[s], rv)
        pltpu.sync_copy(colp_hbm.at[s], cv)

        zeros = jnp.zeros((L,), jnp.float32)
        nf = F // L

        def zb(i, carry):
            r = i // nf
            f = lax.rem(i, nf)
            buf[r, pl.ds(f * L, L)] = zeros
            return carry

        lax.fori_loop(0, B * nf, zb, 0)

        def zc(k, carry):
            pltpu.sync_copy(buf, acc.at[pl.ds(s * ROWS_PER_TILE + k * B, B)])
            return carry

        lax.fori_loop(0, ROWS_PER_TILE // B, zc, 0)
        plsc.subcore_barrier()

        def body(j, carry):
            pltpu.sync_copy(z_ref.at[rv.at[j]], buf)
            pltpu.sync_copy(buf, acc.at[cv.at[j]], add=True)
            return carry

        lax.fori_loop(0, ch, body, 0)
        plsc.subcore_barrier()

        def oc(k, carry):
            off = s * ROWS_PER_TILE + k * B
            pltpu.sync_copy(acc.at[pl.ds(off, B)], out_ref.at[pl.ds(off, B)])
            return carry

        lax.fori_loop(0, ROWS_PER_TILE // B, oc, 0)

    @pl.when(c == 0)
    def _():
        run(zlo_hbm, outlo_hbm)

    @pl.when(c == 1)
    def _():
        run(zhi_hbm, outhi_hbm)


def _make_prop_kernel(ch):
    return pl.kernel(
        _prop_body,
        out_type=[jax.ShapeDtypeStruct((NPAD, F), jnp.float32),
                  jax.ShapeDtypeStruct((NPAD, F), jnp.float32)],
        mesh=plsc.VectorSubcoreMesh(core_axis_name="c", subcore_axis_name="s"),
        compiler_params=pltpu.CompilerParams(needs_layout_passes=False, use_tc_tiling_on_sc=False),
        scratch_types=[
            pltpu.VMEM((ch, B), jnp.int32),
            pltpu.VMEM((ch, B), jnp.int32),
            pltpu.VMEM((B, F), jnp.float32),
            pltpu.VMEM_SHARED((NPAD, F), jnp.float32),
        ],
    )


BLK = 1024


def _build_z_body(degp_ref, x_ref, zlo_ref, zhi_ref):
    i = pl.program_id(0)
    deg = jnp.sum(degp_ref[...], axis=1, keepdims=True) + 1.0
    dis = lax.rsqrt(deg)
    rows = i * BLK + lax.broadcasted_iota(jnp.int32, (BLK, 1), 0)
    discol = jnp.where(rows < N, dis, 0.0)
    zpad = jnp.zeros((BLK, F - 129), jnp.float32)
    zlo_ref[...] = jnp.concatenate(
        [dis * x_ref[:, :128], discol, zpad], axis=1)
    zhi_ref[...] = jnp.concatenate(
        [dis * x_ref[:, 128:], discol * 0.0, zpad], axis=1)


def _build_z(degp2, xp):
    return pl.pallas_call(
        _build_z_body,
        grid=(NPAD // BLK,),
        in_specs=[
            pl.BlockSpec((BLK, NW), lambda i: (i, 0)),
            pl.BlockSpec((BLK, D), lambda i: (i, 0)),
        ],
        out_specs=[
            pl.BlockSpec((BLK, F), lambda i: (i, 0)),
            pl.BlockSpec((BLK, F), lambda i: (i, 0)),
        ],
        out_shape=[jax.ShapeDtypeStruct((NPAD, F), jnp.float32),
                   jax.ShapeDtypeStruct((NPAD, F), jnp.float32)],
    )(degp2, xp)


def _sigmoid(v):
    return 1.0 / (1.0 + jnp.exp(-v))


def _dense_body(degp_ref, x_ref, alo_ref, ahi_ref,
                whp_ref, wlp_ref, wi_ref, bhp_ref, blp_ref, bi_ref,
                gwh_ref, gwl_ref, gwi_ref, gbh_ref, gbl_ref, gbi_ref,
                out_ref):
    deg = jnp.sum(degp_ref[...], axis=1, keepdims=True) + 1.0
    dis = lax.rsqrt(deg)
    invd = 1.0 / deg
    x = x_ref[...]
    agg = jnp.concatenate([alo_ref[:, :128], ahi_ref[:, :128]], axis=1)
    y = dis * agg + invd * x
    srow = dis * alo_ref[:, 128:129] + invd

    h_hp = jnp.dot(x - y, whp_ref[...], preferred_element_type=jnp.float32)
    h_hp = jnp.maximum(h_hp + (1.0 - srow) * bhp_ref[...], 0.0)
    h_lp = jnp.dot(y, wlp_ref[...], preferred_element_type=jnp.float32)
    h_lp = jnp.maximum(h_lp + srow * blp_ref[...], 0.0)
    h_i = jnp.dot(x, wi_ref[...], preferred_element_type=jnp.float32)
    h_i = jnp.maximum(h_i + bi_ref[...], 0.0)

    a_h = _sigmoid(jnp.dot(h_hp, gwh_ref[...],
                           preferred_element_type=jnp.float32) + gbh_ref[...])
    a_l = _sigmoid(jnp.dot(h_lp, gwl_ref[...],
                           preferred_element_type=jnp.float32) + gbl_ref[...])
    a_i = _sigmoid(jnp.dot(h_i, gwi_ref[...],
                           preferred_element_type=jnp.float32) + gbi_ref[...])
    out_ref[...] = a_h * h_hp + a_l * h_lp + a_i * h_i


def _dense(degp2, xp, alo, ahi, W_hp, W_lp, W_i, b_hp, b_lp, b_i,
           wh, wl, wi, bh, bl, bi):
    row_spec = lambda w: pl.BlockSpec((BLK, w), lambda i: (i, 0))
    const_spec = lambda a, b: pl.BlockSpec((a, b), lambda i: (0, 0))
    return pl.pallas_call(
        _dense_body,
        grid=(NPAD // BLK,),
        in_specs=[
            row_spec(NW), row_spec(D), row_spec(F), row_spec(F),
            const_spec(D, D), const_spec(D, D), const_spec(D, D),
            const_spec(1, D), const_spec(1, D), const_spec(1, D),
            const_spec(D, 1), const_spec(D, 1), const_spec(D, 1),
            const_spec(1, 1), const_spec(1, 1), const_spec(1, 1),
        ],
        out_specs=pl.BlockSpec((BLK, D), lambda i: (i, 0)),
        out_shape=jax.ShapeDtypeStruct((NPAD, D), jnp.float32),
    )(degp2, xp, alo, ahi, W_hp, W_lp, W_i,
      b_hp.reshape(1, D), b_lp.reshape(1, D), b_i.reshape(1, D),
      wh, wl, wi, bh.reshape(1, 1), bl.reshape(1, 1), bi.reshape(1, 1))


def kernel(x, edge_index, W_hp, b_hp, W_lp, b_lp, W_i, b_i,
           wh, bh, wl, bl, wi, bi):
    e = edge_index.shape[1]
    ch = -(-e // (NS * B))
    epad = NS * ch * B
    row = edge_index[0].astype(jnp.int32)
    col = edge_index[1].astype(jnp.int32)
    rowp = jnp.where(row == col, ZROW, row)
    rowp_full = jnp.full((epad,), ZROW, jnp.int32).at[:e].set(rowp)
    colp_full = jnp.zeros((epad,), jnp.int32).at[:e].set(col)
    rowp3 = rowp_full.reshape(NS, ch, B)
    colp3 = colp_full.reshape(NS, ch, B)
    rowp_d = rowp_full.reshape(NW, epad // NW)
    colp_d = colp_full.reshape(NW, epad // NW)

    xp = jnp.zeros((NPAD, D), jnp.float32).at[:N].set(x)

    degp = _make_deg_kernel(epad // NW)(rowp_d, colp_d)
    degp2 = degp.T
    zlo, zhi = _build_z(degp2, xp)
    alo, ahi = _make_prop_kernel(ch)(rowp3, colp3, zlo, zhi)
    out = _dense(degp2, xp, alo, ahi, W_hp, W_lp, W_i, b_hp, b_lp, b_i,
                 wh, wl, wi, bh, bl, bi)
    return out[:N]


# speedup vs baseline: 21.7680x; 1.3761x over previous
import functools

import jax
import jax.numpy as jnp
from jax import lax
from jax.experimental import pallas as pl
from jax.experimental.pallas import tpu as pltpu
from jax.experimental.pallas import tpu_sc as plsc

N = 10000
D = 256
NC, NS, L = 2, 16, 16
NW = NC * NS
NPAD = 10240
ZROW = N
B = 64
F = 144
ROWS_PER_TILE = NPAD // NS


def _deg_body(rowp_hbm, colp_hbm, out_hbm, rv, cv, dloc):
    c = lax.axis_index("c")
    s = lax.axis_index("s")
    wid = s * NC + c
    ed = rv.shape[0]
    pltpu.sync_copy(rowp_hbm.at[wid], rv)
    pltpu.sync_copy(colp_hbm.at[wid], cv)

    zeros = jnp.zeros((L,), jnp.float32)

    def zb(i, carry):
        dloc[pl.ds(i * L, L)] = zeros
        return carry

    lax.fori_loop(0, NPAD // L, zb, 0)

    ones = jnp.ones((L,), jnp.float32)

    def body(i, carry):
        r = rv[pl.ds(i * L, L)]
        cc = cv[pl.ds(i * L, L)]
        plsc.addupdate_scatter(dloc, [cc], ones, mask=r != ZROW)
        return carry

    lax.fori_loop(0, ed // L, body, 0)

    pltpu.sync_copy(dloc, out_hbm.at[wid])


def _make_deg_kernel(ed):
    return pl.kernel(
        _deg_body,
        out_type=jax.ShapeDtypeStruct((NW, NPAD), jnp.float32),
        mesh=plsc.VectorSubcoreMesh(core_axis_name="c", subcore_axis_name="s"),
        compiler_params=pltpu.CompilerParams(needs_layout_passes=False, use_tc_tiling_on_sc=False),
        scratch_types=[
            pltpu.VMEM((ed,), jnp.int32),
            pltpu.VMEM((ed,), jnp.int32),
            pltpu.VMEM((NPAD,), jnp.float32),
        ],
    )


NDB = 2


def _prop_body(rowp_hbm, colp_hbm, zlo_hbm, zhi_hbm, outlo_hbm, outhi_hbm,
               rv, cv, bufs, sems, acc):
    c = lax.axis_index("c")
    s = lax.axis_index("s")
    ch = rv.shape[0]

    def run(z_ref, out_ref):
        pltpu.sync_copy(rowp_hbm.at[s], rv)
        pltpu.sync_copy(colp_hbm.at[s], cv)

        zeros = jnp.zeros((L,), jnp.float32)
        nf = F // L

        def zb(i, carry):
            r = i // nf
            f = lax.rem(i, nf)
            bufs[0, r, pl.ds(f * L, L)] = zeros
            return carry

        lax.fori_loop(0, B * nf, zb, 0)

        def zc(k, carry):
            pltpu.sync_copy(bufs.at[0],
                            acc.at[pl.ds(s * ROWS_PER_TILE + k * B, B)])
            return carry

        lax.fori_loop(0, ROWS_PER_TILE // B, zc, 0)
        plsc.subcore_barrier()

        for b in range(NDB):
            pltpu.async_copy(z_ref.at[rv.at[b]], bufs.at[b], sems[b])

        def body(i, carry):
            for b in range(NDB):
                j = i * NDB + b

                @pl.when(j < ch)
                def _():
                    pltpu.make_async_copy(
                        z_ref.at[rv.at[j]], bufs.at[b], sems[b]).wait()
                    pltpu.sync_copy(bufs.at[b], acc.at[cv.at[j]], add=True)

                @pl.when(j + NDB < ch)
                def _():
                    pltpu.async_copy(
                        z_ref.at[rv.at[j + NDB]], bufs.at[b], sems[b])
            return carry

        lax.fori_loop(0, (ch + NDB - 1) // NDB, body, 0)
        plsc.subcore_barrier()

        def oc(k, carry):
            off = s * ROWS_PER_TILE + k * B
            pltpu.sync_copy(acc.at[pl.ds(off, B)], out_ref.at[pl.ds(off, B)])
            return carry

        lax.fori_loop(0, ROWS_PER_TILE // B, oc, 0)

    @pl.when(c == 0)
    def _():
        run(zlo_hbm, outlo_hbm)

    @pl.when(c == 1)
    def _():
        run(zhi_hbm, outhi_hbm)


def _make_prop_kernel(ch):
    return pl.kernel(
        _prop_body,
        out_type=[jax.ShapeDtypeStruct((NPAD, F), jnp.float32),
                  jax.ShapeDtypeStruct((NPAD, F), jnp.float32)],
        mesh=plsc.VectorSubcoreMesh(core_axis_name="c", subcore_axis_name="s"),
        compiler_params=pltpu.CompilerParams(needs_layout_passes=False, use_tc_tiling_on_sc=False),
        scratch_types=[
            pltpu.VMEM((ch, B), jnp.int32),
            pltpu.VMEM((ch, B), jnp.int32),
            pltpu.VMEM((NDB, B, F), jnp.float32),
            [pltpu.SemaphoreType.DMA] * NDB,
            pltpu.VMEM_SHARED((NPAD, F), jnp.float32),
        ],
    )


BLK = 1024


def _build_z_body(degp_ref, x_ref, zlo_ref, zhi_ref):
    i = pl.program_id(0)
    deg = jnp.sum(degp_ref[...], axis=1, keepdims=True) + 1.0
    dis = lax.rsqrt(deg)
    rows = i * BLK + lax.broadcasted_iota(jnp.int32, (BLK, 1), 0)
    discol = jnp.where(rows < N, dis, 0.0)
    zpad = jnp.zeros((BLK, F - 129), jnp.float32)
    zlo_ref[...] = jnp.concatenate(
        [dis * x_ref[:, :128], discol, zpad], axis=1)
    zhi_ref[...] = jnp.concatenate(
        [dis * x_ref[:, 128:], discol * 0.0, zpad], axis=1)


def _build_z(degp2, xp):
    return pl.pallas_call(
        _build_z_body,
        grid=(NPAD // BLK,),
        in_specs=[
            pl.BlockSpec((BLK, NW), lambda i: (i, 0)),
            pl.BlockSpec((BLK, D), lambda i: (i, 0)),
        ],
        out_specs=[
            pl.BlockSpec((BLK, F), lambda i: (i, 0)),
            pl.BlockSpec((BLK, F), lambda i: (i, 0)),
        ],
        out_shape=[jax.ShapeDtypeStruct((NPAD, F), jnp.float32),
                   jax.ShapeDtypeStruct((NPAD, F), jnp.float32)],
    )(degp2, xp)


def _sigmoid(v):
    return 1.0 / (1.0 + jnp.exp(-v))


def _dense_body(degp_ref, x_ref, alo_ref, ahi_ref,
                whp_ref, wlp_ref, wi_ref, bhp_ref, blp_ref, bi_ref,
                gwh_ref, gwl_ref, gwi_ref, gbh_ref, gbl_ref, gbi_ref,
                out_ref):
    deg = jnp.sum(degp_ref[...], axis=1, keepdims=True) + 1.0
    dis = lax.rsqrt(deg)
    invd = 1.0 / deg
    x = x_ref[...]
    agg = jnp.concatenate([alo_ref[:, :128], ahi_ref[:, :128]], axis=1)
    y = dis * agg + invd * x
    srow = dis * alo_ref[:, 128:129] + invd

    h_hp = jnp.dot(x - y, whp_ref[...], preferred_element_type=jnp.float32)
    h_hp = jnp.maximum(h_hp + (1.0 - srow) * bhp_ref[...], 0.0)
    h_lp = jnp.dot(y, wlp_ref[...], preferred_element_type=jnp.float32)
    h_lp = jnp.maximum(h_lp + srow * blp_ref[...], 0.0)
    h_i = jnp.dot(x, wi_ref[...], preferred_element_type=jnp.float32)
    h_i = jnp.maximum(h_i + bi_ref[...], 0.0)

    a_h = _sigmoid(jnp.dot(h_hp, gwh_ref[...],
                           preferred_element_type=jnp.float32) + gbh_ref[...])
    a_l = _sigmoid(jnp.dot(h_lp, gwl_ref[...],
                           preferred_element_type=jnp.float32) + gbl_ref[...])
    a_i = _sigmoid(jnp.dot(h_i, gwi_ref[...],
                           preferred_element_type=jnp.float32) + gbi_ref[...])
    out_ref[...] = a_h * h_hp + a_l * h_lp + a_i * h_i


def _dense(degp2, xp, alo, ahi, W_hp, W_lp, W_i, b_hp, b_lp, b_i,
           wh, wl, wi, bh, bl, bi):
    row_spec = lambda w: pl.BlockSpec((BLK, w), lambda i: (i, 0))
    const_spec = lambda a, b: pl.BlockSpec((a, b), lambda i: (0, 0))
    return pl.pallas_call(
        _dense_body,
        grid=(NPAD // BLK,),
        in_specs=[
            row_spec(NW), row_spec(D), row_spec(F), row_spec(F),
            const_spec(D, D), const_spec(D, D), const_spec(D, D),
            const_spec(1, D), const_spec(1, D), const_spec(1, D),
            const_spec(D, 1), const_spec(D, 1), const_spec(D, 1),
            const_spec(1, 1), const_spec(1, 1), const_spec(1, 1),
        ],
        out_specs=pl.BlockSpec((BLK, D), lambda i: (i, 0)),
        out_shape=jax.ShapeDtypeStruct((NPAD, D), jnp.float32),
    )(degp2, xp, alo, ahi, W_hp, W_lp, W_i,
      b_hp.reshape(1, D), b_lp.reshape(1, D), b_i.reshape(1, D),
      wh, wl, wi, bh.reshape(1, 1), bl.reshape(1, 1), bi.reshape(1, 1))


def kernel(x, edge_index, W_hp, b_hp, W_lp, b_lp, W_i, b_i,
           wh, bh, wl, bl, wi, bi):
    e = edge_index.shape[1]
    ch = -(-e // (NS * B))
    epad = NS * ch * B
    row = edge_index[0].astype(jnp.int32)
    col = edge_index[1].astype(jnp.int32)
    rowp = jnp.where(row == col, ZROW, row)
    rowp_full = jnp.full((epad,), ZROW, jnp.int32).at[:e].set(rowp)
    colp_full = jnp.zeros((epad,), jnp.int32).at[:e].set(col)
    rowp3 = rowp_full.reshape(NS, ch, B)
    colp3 = colp_full.reshape(NS, ch, B)
    rowp_d = rowp_full.reshape(NW, epad // NW)
    colp_d = colp_full.reshape(NW, epad // NW)

    xp = jnp.zeros((NPAD, D), jnp.float32).at[:N].set(x)

    degp = _make_deg_kernel(epad // NW)(rowp_d, colp_d)
    degp2 = degp.T
    zlo, zhi = _build_z(degp2, xp)
    alo, ahi = _make_prop_kernel(ch)(rowp3, colp3, zlo, zhi)
    out = _dense(degp2, xp, alo, ahi, W_hp, W_lp, W_i, b_hp, b_lp, b_i,
                 wh, wl, wi, bh, bl, bi)
    return out[:N]


# speedup vs baseline: 23.5151x; 1.0803x over previous
import functools

import jax
import jax.numpy as jnp
from jax import lax
from jax.experimental import pallas as pl
from jax.experimental.pallas import tpu as pltpu
from jax.experimental.pallas import tpu_sc as plsc

N = 10000
D = 256
NC, NS, L = 2, 16, 16
NW = NC * NS
NPAD = 10240
ZROW = N
B = 32
F = 144
ROWS_PER_TILE = NPAD // NS


def _deg_body(rowp_hbm, colp_hbm, out_hbm, rv, cv, dloc):
    c = lax.axis_index("c")
    s = lax.axis_index("s")
    wid = s * NC + c
    ed = rv.shape[0]
    pltpu.sync_copy(rowp_hbm.at[wid], rv)
    pltpu.sync_copy(colp_hbm.at[wid], cv)

    zeros = jnp.zeros((L,), jnp.float32)

    def zb(i, carry):
        dloc[pl.ds(i * L, L)] = zeros
        return carry

    lax.fori_loop(0, NPAD // L, zb, 0)

    ones = jnp.ones((L,), jnp.float32)

    def body(i, carry):
        r = rv[pl.ds(i * L, L)]
        cc = cv[pl.ds(i * L, L)]
        plsc.addupdate_scatter(dloc, [cc], ones, mask=r != cc)
        return carry

    lax.fori_loop(0, ed // L, body, 0)

    pltpu.sync_copy(dloc, out_hbm.at[wid])


def _make_deg_kernel(ed):
    return pl.kernel(
        _deg_body,
        out_type=jax.ShapeDtypeStruct((NW, NPAD), jnp.float32),
        mesh=plsc.VectorSubcoreMesh(core_axis_name="c", subcore_axis_name="s"),
        compiler_params=pltpu.CompilerParams(needs_layout_passes=False, use_tc_tiling_on_sc=False),
        scratch_types=[
            pltpu.VMEM((ed,), jnp.int32),
            pltpu.VMEM((ed,), jnp.int32),
            pltpu.VMEM((NPAD,), jnp.float32),
        ],
    )


NDB = 4


def _prop_body(row_hbm, col_hbm, zlo_hbm, zhi_hbm, outlo_hbm, outhi_hbm,
               rv, cv, bufs, gsems, ssems, acc):
    c = lax.axis_index("c")
    s = lax.axis_index("s")
    ch = rv.shape[0]

    def remap(j):
        for k in range(B // L):
            r = rv[j, pl.ds(k * L, L)]
            cc = cv[j, pl.ds(k * L, L)]
            rv[j, pl.ds(k * L, L)] = jnp.where(r == cc, ZROW, r)

    def run(z_ref, out_ref):
        pltpu.sync_copy(row_hbm.at[s], rv)
        pltpu.sync_copy(col_hbm.at[s], cv)

        zeros = jnp.zeros((L,), jnp.float32)
        nf = F // L

        def zb(i, carry):
            r = i // nf
            f = lax.rem(i, nf)
            bufs[0, r, pl.ds(f * L, L)] = zeros
            return carry

        lax.fori_loop(0, B * nf, zb, 0)

        def zc(k, carry):
            pltpu.sync_copy(bufs.at[0],
                            acc.at[pl.ds(s * ROWS_PER_TILE + k * B, B)])
            return carry

        lax.fori_loop(0, ROWS_PER_TILE // B, zc, 0)
        plsc.subcore_barrier()

        for b in range(NDB):
            remap(b)
            pltpu.async_copy(z_ref.at[rv.at[b]], bufs.at[b], gsems[b])

        def body(i, carry):
            for b in range(NDB):
                j = i * NDB + b
                bp = (b - 1) % NDB

                @pl.when(j < ch)
                def _():
                    pltpu.make_async_copy(
                        z_ref.at[rv.at[j]], bufs.at[b], gsems[b]).wait()
                    pltpu.sync_copy(bufs.at[b], acc.at[cv.at[j]], add=True)

                jn = j + NDB - 1

                @pl.when((j >= 1) & (jn < ch))
                def _():
                    remap(jn)
                    pltpu.async_copy(z_ref.at[rv.at[jn]], bufs.at[bp],
                                     gsems[bp])
            return carry

        lax.fori_loop(0, (ch + NDB - 1) // NDB, body, 0)
        plsc.subcore_barrier()

        def oc(k, carry):
            off = s * ROWS_PER_TILE + k * B
            pltpu.sync_copy(acc.at[pl.ds(off, B)], out_ref.at[pl.ds(off, B)])
            return carry

        lax.fori_loop(0, ROWS_PER_TILE // B, oc, 0)

    @pl.when(c == 0)
    def _():
        run(zlo_hbm, outlo_hbm)

    @pl.when(c == 1)
    def _():
        run(zhi_hbm, outhi_hbm)


def _make_prop_kernel(ch):
    return pl.kernel(
        _prop_body,
        out_type=[jax.ShapeDtypeStruct((NPAD, F), jnp.float32),
                  jax.ShapeDtypeStruct((NPAD, F), jnp.float32)],
        mesh=plsc.VectorSubcoreMesh(core_axis_name="c", subcore_axis_name="s"),
        compiler_params=pltpu.CompilerParams(needs_layout_passes=False, use_tc_tiling_on_sc=False),
        scratch_types=[
            pltpu.VMEM((ch, B), jnp.int32),
            pltpu.VMEM((ch, B), jnp.int32),
            pltpu.VMEM((NDB, B, F), jnp.float32),
            [pltpu.SemaphoreType.DMA] * NDB,
            [pltpu.SemaphoreType.DMA] * NDB,
            pltpu.VMEM_SHARED((NPAD, F), jnp.float32),
        ],
    )


BLK = 1024


def _build_z_body(degp_ref, x_ref, zlo_ref, zhi_ref):
    i = pl.program_id(0)
    deg = jnp.sum(degp_ref[...], axis=1, keepdims=True) + 1.0
    dis = lax.rsqrt(deg)
    rows = i * BLK + lax.broadcasted_iota(jnp.int32, (BLK, 1), 0)
    discol = jnp.where(rows < N, dis, 0.0)
    zpad = jnp.zeros((BLK, F - 129), jnp.float32)
    zlo_ref[...] = jnp.concatenate(
        [dis * x_ref[:, :128], discol, zpad], axis=1)
    zhi_ref[...] = jnp.concatenate(
        [dis * x_ref[:, 128:], discol * 0.0, zpad], axis=1)


def _build_z(degp2, xp):
    return pl.pallas_call(
        _build_z_body,
        grid=(NPAD // BLK,),
        in_specs=[
            pl.BlockSpec((BLK, NW), lambda i: (i, 0)),
            pl.BlockSpec((BLK, D), lambda i: (i, 0)),
        ],
        out_specs=[
            pl.BlockSpec((BLK, F), lambda i: (i, 0)),
            pl.BlockSpec((BLK, F), lambda i: (i, 0)),
        ],
        out_shape=[jax.ShapeDtypeStruct((NPAD, F), jnp.float32),
                   jax.ShapeDtypeStruct((NPAD, F), jnp.float32)],
    )(degp2, xp)


def _sigmoid(v):
    return 1.0 / (1.0 + jnp.exp(-v))


def _dense_body(degp_ref, x_ref, alo_ref, ahi_ref,
                whp_ref, wlp_ref, wi_ref, bhp_ref, blp_ref, bi_ref,
                gwh_ref, gwl_ref, gwi_ref, gbh_ref, gbl_ref, gbi_ref,
                out_ref):
    deg = jnp.sum(degp_ref[...], axis=1, keepdims=True) + 1.0
    dis = lax.rsqrt(deg)
    invd = 1.0 / deg
    x = x_ref[...]
    agg = jnp.concatenate([alo_ref[:, :128], ahi_ref[:, :128]], axis=1)
    y = dis * agg + invd * x
    srow = dis * alo_ref[:, 128:129] + invd

    h_hp = jnp.dot(x - y, whp_ref[...], preferred_element_type=jnp.float32)
    h_hp = jnp.maximum(h_hp + (1.0 - srow) * bhp_ref[...], 0.0)
    h_lp = jnp.dot(y, wlp_ref[...], preferred_element_type=jnp.float32)
    h_lp = jnp.maximum(h_lp + srow * blp_ref[...], 0.0)
    h_i = jnp.dot(x, wi_ref[...], preferred_element_type=jnp.float32)
    h_i = jnp.maximum(h_i + bi_ref[...], 0.0)

    a_h = _sigmoid(jnp.dot(h_hp, gwh_ref[...],
                           preferred_element_type=jnp.float32) + gbh_ref[...])
    a_l = _sigmoid(jnp.dot(h_lp, gwl_ref[...],
                           preferred_element_type=jnp.float32) + gbl_ref[...])
    a_i = _sigmoid(jnp.dot(h_i, gwi_ref[...],
                           preferred_element_type=jnp.float32) + gbi_ref[...])
    out_ref[...] = a_h * h_hp + a_l * h_lp + a_i * h_i


def _dense(degp2, xp, alo, ahi, W_hp, W_lp, W_i, b_hp, b_lp, b_i,
           wh, wl, wi, bh, bl, bi):
    row_spec = lambda w: pl.BlockSpec((BLK, w), lambda i: (i, 0))
    const_spec = lambda a, b: pl.BlockSpec((a, b), lambda i: (0, 0))
    return pl.pallas_call(
        _dense_body,
        grid=(NPAD // BLK,),
        in_specs=[
            row_spec(NW), row_spec(D), row_spec(F), row_spec(F),
            const_spec(D, D), const_spec(D, D), const_spec(D, D),
            const_spec(1, D), const_spec(1, D), const_spec(1, D),
            const_spec(D, 1), const_spec(D, 1), const_spec(D, 1),
            const_spec(1, 1), const_spec(1, 1), const_spec(1, 1),
        ],
        out_specs=pl.BlockSpec((BLK, D), lambda i: (i, 0)),
        out_shape=jax.ShapeDtypeStruct((NPAD, D), jnp.float32),
    )(degp2, xp, alo, ahi, W_hp, W_lp, W_i,
      b_hp.reshape(1, D), b_lp.reshape(1, D), b_i.reshape(1, D),
      wh, wl, wi, bh.reshape(1, 1), bl.reshape(1, 1), bi.reshape(1, 1))


def kernel(x, edge_index, W_hp, b_hp, W_lp, b_lp, W_i, b_i,
           wh, bh, wl, bl, wi, bi):
    e = edge_index.shape[1]
    ch = -(-e // (NS * B))
    epad = NS * ch * B
    row_full = jnp.zeros((epad,), jnp.int32).at[:e].set(
        edge_index[0].astype(jnp.int32))
    col_full = jnp.zeros((epad,), jnp.int32).at[:e].set(
        edge_index[1].astype(jnp.int32))
    rowp3 = row_full.reshape(NS, ch, B)
    colp3 = col_full.reshape(NS, ch, B)
    rowp_d = row_full.reshape(NW, epad // NW)
    colp_d = col_full.reshape(NW, epad // NW)

    xp = jnp.zeros((NPAD, D), jnp.float32).at[:N].set(x)

    degp = _make_deg_kernel(epad // NW)(rowp_d, colp_d)
    degp2 = degp.T
    zlo, zhi = _build_z(degp2, xp)
    alo, ahi = _make_prop_kernel(ch)(rowp3, colp3, zlo, zhi)
    out = _dense(degp2, xp, alo, ahi, W_hp, W_lp, W_i, b_hp, b_lp, b_i,
                 wh, wl, wi, bh, bl, bi)
    return out[:N]
